# baseline (speedup 1.0000x reference)
import jax
import jax.numpy as jnp
from jax.experimental import pallas as pl
from jax.experimental.pallas import tpu as pltpu

BS = 7
H = W = 56
MH = MW = 50


def _dilate(m):
    K = m.shape[0]
    zH = jnp.zeros((K, H - MH, MW), dtype=m.dtype)
    mp = jnp.concatenate([m, zH], axis=1)
    zW = jnp.zeros((K, H, W - MW), dtype=m.dtype)
    mp = jnp.concatenate([mp, zW], axis=2)

    def shift_down(a, s, axis):
        if axis == 1:
            z = jnp.zeros((K, s, W), dtype=a.dtype)
            return jnp.concatenate([z, a], axis=1)[:, :H, :]
        z = jnp.zeros((K, H, s), dtype=a.dtype)
        return jnp.concatenate([z, a], axis=2)[:, :, :W]

    acc = mp
    for s in (1, 2, 3):
        acc = jnp.maximum(acc, shift_down(acc, s, 1))
    for s in (1, 2, 3):
        acc = jnp.maximum(acc, shift_down(acc, s, 2))
    return acc


def _count_body(mask_ref, cnt_ref):
    i = pl.program_id(0)

    @pl.when(i == 0)
    def _():
        cnt_ref[0, 0] = 0.0

    d = _dilate(mask_ref[...])
    cnt_ref[0, 0] += jnp.sum(d)


def _apply_body(cnt_ref, x_ref, mask_ref, out_ref):
    count_m = jnp.float32(x_ref.shape[0] * H * W * pl.num_programs(0))
    scale = count_m / (count_m - cnt_ref[0, 0])
    d = _dilate(mask_ref[...])
    out_ref[...] = jnp.where(d > 0.0, 0.0, x_ref[...] * scale)


def kernel(x, mask):
    B, C, _, _ = x.shape
    N = B * C
    xv = x.reshape(N, H, W)
    mv = mask.reshape(N, MH, MW)

    K = 128
    grid = (N // K,)

    cnt = pl.pallas_call(
        _count_body,
        grid=grid,
        in_specs=[pl.BlockSpec((K, MH, MW), lambda i: (i, 0, 0))],
        out_specs=pl.BlockSpec(
            (1, 1), lambda i: (0, 0), memory_space=pltpu.SMEM
        ),
        out_shape=jax.ShapeDtypeStruct((1, 1), jnp.float32),
    )(mv)

    out = pl.pallas_call(
        _apply_body,
        grid=grid,
        in_specs=[
            pl.BlockSpec(memory_space=pltpu.SMEM),
            pl.BlockSpec((K, H, W), lambda i: (i, 0, 0)),
            pl.BlockSpec((K, MH, MW), lambda i: (i, 0, 0)),
        ],
        out_specs=pl.BlockSpec((K, H, W), lambda i: (i, 0, 0)),
        out_shape=jax.ShapeDtypeStruct((N, H, W), jnp.float32),
    )(cnt, xv, mv)

    return out.reshape(B, C, H, W)
